# baseline (speedup 1.0000x reference)
import jax
import jax.numpy as jnp
from jax import lax
from jax.experimental import pallas as pl
from jax.experimental.pallas import tpu as pltpu
from jax.experimental.pallas import tpu_sc as plsc

N = 10000
E = 320000
D = 128
K_ITERS = 10
ALPHA = 0.1

NC = 2
NS = 16
NW = NC * NS
CHUNK = 64
NCHUNK = 160
EPT = NCHUNK * CHUNK
RPT = 626
ACC_ROWS = NS * RPT
TRASH = N

ROW_BLK = 1000


NBUF = 4
DEFER = 2
QCH = NCHUNK // 4
ZROWS = RPT // 9


def _prop_body(src_hbm, dst_hbm, v_hbm, o_hbm,
               src_v, dst_v, stage0, stage1, stage2, stage3, acc, gsem, ssem):
    stage = (stage0, stage1, stage2, stage3)
    c = lax.axis_index("c")
    s = lax.axis_index("s")
    w = c * NS + s
    base = s * RPT
    ebase = w * NCHUNK

    zf = jnp.zeros((16,), jnp.float32)

    def _zrow(i, carry):
        for jj in range(D // 16):
            stage0[i, pl.ds(jj * 16, 16)] = zf
        return carry

    lax.fori_loop(0, CHUNK, _zrow, 0)
    for k in range((RPT + CHUNK - 1) // CHUNK):
        nr = min(CHUNK, RPT - k * CHUNK)
        pltpu.sync_copy(stage0.at[pl.ds(0, nr)],
                        acc.at[pl.ds(base + k * CHUNK, nr)])
    plsc.subcore_barrier()

    def _fire_g(j, b):
        pltpu.async_copy(v_hbm.at[src_v.at[j]], stage[b], gsem)

    def _wait_g(j, b):
        pltpu.make_async_copy(v_hbm.at[src_v.at[j]], stage[b], gsem).wait()

    def _fire_s(j, b):
        pltpu.async_copy(stage[b], acc.at[dst_v.at[j]], ssem, add=True)

    def _wait_s(b):
        pltpu.make_async_copy(stage[b], acc.at[dst_v.at[0]], ssem).wait()

    def _quarter(off):
        pltpu.sync_copy(src_hbm.at[pl.ds(ebase + off, QCH)], src_v)
        pltpu.sync_copy(dst_hbm.at[pl.ds(ebase + off, QCH)], dst_v)
        for b in range(DEFER):
            _fire_g(b, b)
        for j in range(QCH):
            b = j % NBUF
            _wait_g(j, b)
            _fire_s(j, b)
            jn = j + DEFER
            if jn < QCH:
                if jn >= NBUF:
                    _wait_s(jn % NBUF)
                _fire_g(jn, jn % NBUF)
        for j in range(QCH - NBUF, QCH):
            _wait_s(j % NBUF)

    for q in range(4):
        _quarter(q * QCH)

    plsc.subcore_barrier()
    pltpu.sync_copy(acc.at[pl.ds(base, RPT)], o_hbm.at[w])


_sc_prop = pl.kernel(
    _prop_body,
    out_type=jax.ShapeDtypeStruct((NW, RPT, D), jnp.float32),
    mesh=plsc.VectorSubcoreMesh(core_axis_name="c", subcore_axis_name="s"),
    scratch_types=[
        pltpu.VMEM((QCH, CHUNK), jnp.int32),
        pltpu.VMEM((QCH, CHUNK), jnp.int32),
        pltpu.VMEM((CHUNK, D), jnp.float32),
        pltpu.VMEM((CHUNK, D), jnp.float32),
        pltpu.VMEM((CHUNK, D), jnp.float32),
        pltpu.VMEM((CHUNK, D), jnp.float32),
        pltpu.VMEM_SHARED((ACC_ROWS, D), jnp.float32),
        pltpu.SemaphoreType.DMA,
        pltpu.SemaphoreType.DMA,
    ],
)


def _rspec(ncols=D):
    return pl.BlockSpec((ROW_BLK, ncols), lambda i: (i, 0))


def _dis_body(deg_ref, dis_ref, dis2_ref, dinv_ref):
    d = deg_ref[...] + 1.0
    r = lax.rsqrt(d)
    dis_ref[...] = r
    dis2_ref[...] = 1.0 / d
    dinv_ref[...] = jnp.sqrt(d)


def _dis_call(deg):
    return pl.pallas_call(
        _dis_body,
        grid=(N // ROW_BLK,),
        in_specs=[_rspec(1)],
        out_specs=[_rspec(1)] * 3,
        out_shape=[jax.ShapeDtypeStruct((N, 1), jnp.float32)] * 3,
    )(deg)


def _mm1_body(x_ref, w_ref, dis_ref, o_ref):
    t = jnp.dot(x_ref[...], w_ref[...], preferred_element_type=jnp.float32)
    o_ref[...] = t * dis_ref[...]


def _mm1_call(x, W1, dis):
    return pl.pallas_call(
        _mm1_body,
        grid=(N // ROW_BLK,),
        in_specs=[_rspec(), pl.BlockSpec((D, D), lambda i: (0, 0)), _rspec(1)],
        out_specs=_rspec(),
        out_shape=jax.ShapeDtypeStruct((N, D), jnp.float32),
    )(x, W1, dis)


def _first_body(a0_ref, a1_ref, t_ref, dis_ref, b1_ref, v_ref, w_ref):
    dis = dis_ref[...]
    h = jnp.maximum(dis * (a0_ref[...] + a1_ref[...] + t_ref[...])
                    + b1_ref[...], 0.0)
    v = dis * h
    v_ref[...] = v
    w_ref[...] = ALPHA * v


def _first_call(a0, a1, t, dis, b1):
    return pl.pallas_call(
        _first_body,
        grid=(N // ROW_BLK,),
        in_specs=[_rspec(), _rspec(), _rspec(), _rspec(1),
                  pl.BlockSpec((1, D), lambda i: (0, 0))],
        out_specs=[_rspec()] * 2,
        out_shape=[jax.ShapeDtypeStruct((N, D), jnp.float32)] * 2,
    )(a0, a1, t, dis, b1)


def _mix_body(a0_ref, a1_ref, v_ref, w_ref, dis2_ref, o_ref):
    f = (1.0 - ALPHA) * dis2_ref[...]
    o_ref[...] = f * (a0_ref[...] + a1_ref[...] + v_ref[...]) + w_ref[...]


def _mix_call(a0, a1, v, w, dis2):
    return pl.pallas_call(
        _mix_body,
        grid=(N // ROW_BLK,),
        in_specs=[_rspec(), _rspec(), _rspec(), _rspec(), _rspec(1)],
        out_specs=_rspec(),
        out_shape=jax.ShapeDtypeStruct((N, D), jnp.float32),
    )(a0, a1, v, w, dis2)


def _mm2_body(v_ref, w_ref, dinv_ref, dis_ref, o_ref):
    h = dinv_ref[...] * v_ref[...]
    g = jnp.dot(h, w_ref[...], preferred_element_type=jnp.float32)
    o_ref[...] = dis_ref[...] * g


def _mm2_call(v, W2, dinv, dis):
    return pl.pallas_call(
        _mm2_body,
        grid=(N // ROW_BLK,),
        in_specs=[_rspec(), pl.BlockSpec((D, D), lambda i: (0, 0)),
                  _rspec(1), _rspec(1)],
        out_specs=_rspec(),
        out_shape=jax.ShapeDtypeStruct((N, D), jnp.float32),
    )(v, W2, dinv, dis)


def _out_body(a0_ref, a1_ref, g_ref, dis_ref, b2_ref, o_ref):
    o = dis_ref[...] * (a0_ref[...] + a1_ref[...] + g_ref[...])
    o_ref[...] = o + b2_ref[...]


def _out_call(a0, a1, g, dis, b2):
    return pl.pallas_call(
        _out_body,
        grid=(N // ROW_BLK,),
        in_specs=[_rspec(), _rspec(), _rspec(), _rspec(1),
                  pl.BlockSpec((1, D), lambda i: (0, 0))],
        out_specs=_rspec(),
        out_shape=jax.ShapeDtypeStruct((N, D), jnp.float32),
    )(a0, a1, g, dis, b2)


def _halves(o):
    a0 = o[:NS].reshape(ACC_ROWS, D)[:N]
    a1 = o[NS:].reshape(ACC_ROWS, D)[:N]
    return a0, a1


def kernel(x, edge_index, W1, b1, W2, b2):
    pad = EPT * NW - E
    src = jnp.concatenate([edge_index[0], jnp.zeros((pad,), jnp.int32)])
    dst = jnp.concatenate([edge_index[1], jnp.full((pad,), TRASH, jnp.int32)])
    src_g = src.reshape(NW * NCHUNK, CHUNK)
    dst_g = dst.reshape(NW * NCHUNK, CHUNK)
    b1r = b1.reshape(1, D)
    b2r = b2.reshape(1, D)

    ones = jnp.ones((N, D), jnp.float32)
    d0, d1 = _halves(_sc_prop(src_g, dst_g, ones))
    deg = d0[:, 0:1] + d1[:, 0:1]
    dis, dis2, dinv = _dis_call(deg)

    t = _mm1_call(x, W1, dis)
    a0, a1 = _halves(_sc_prop(src_g, dst_g, t))
    v, w = _first_call(a0, a1, t, dis, b1r)

    for _ in range(K_ITERS):
        a0, a1 = _halves(_sc_prop(src_g, dst_g, v))
        v = _mix_call(a0, a1, v, w, dis2)

    g = _mm2_call(v, W2, dinv, dis)
    a0, a1 = _halves(_sc_prop(src_g, dst_g, g))
    return _out_call(a0, a1, g, dis, b2r)


# speedup vs baseline: 1.0735x; 1.0735x over previous
import jax
import jax.numpy as jnp
from jax import lax
from jax.experimental import pallas as pl
from jax.experimental.pallas import tpu as pltpu
from jax.experimental.pallas import tpu_sc as plsc

N = 10000
E = 320000
D = 128
K_ITERS = 10
ALPHA = 0.1

NC = 2
NS = 16
NW = NC * NS
CHUNK = 80
NCHUNK = 128
EPT = NCHUNK * CHUNK
RPT = 626
ACC_ROWS = NS * RPT
TRASH = N

ROW_BLK = 1000


NBUF = 3
DEFER = 2
QCH = NCHUNK // 4
ZROWS = RPT // 9


def _prop_body(src_hbm, dst_hbm, v_hbm, o_hbm,
               src_v, dst_v, stage0, stage1, stage2, acc, gsem, ssem):
    stage = (stage0, stage1, stage2)
    c = lax.axis_index("c")
    s = lax.axis_index("s")
    w = c * NS + s
    base = s * RPT
    ebase = w * NCHUNK

    zf = jnp.zeros((16,), jnp.float32)

    def _zrow(i, carry):
        for jj in range(D // 16):
            stage0[i, pl.ds(jj * 16, 16)] = zf
        return carry

    lax.fori_loop(0, CHUNK, _zrow, 0)
    for k in range((RPT + CHUNK - 1) // CHUNK):
        nr = min(CHUNK, RPT - k * CHUNK)
        pltpu.sync_copy(stage0.at[pl.ds(0, nr)],
                        acc.at[pl.ds(base + k * CHUNK, nr)])
    plsc.subcore_barrier()

    def _fire_g(j, b):
        pltpu.async_copy(v_hbm.at[src_v.at[j]], stage[b], gsem)

    def _wait_g(j, b):
        pltpu.make_async_copy(v_hbm.at[src_v.at[j]], stage[b], gsem).wait()

    def _fire_s(j, b):
        pltpu.async_copy(stage[b], acc.at[dst_v.at[j]], ssem, add=True)

    def _wait_s(b):
        pltpu.make_async_copy(stage[b], acc.at[dst_v.at[0]], ssem).wait()

    def _quarter(off):
        pltpu.sync_copy(src_hbm.at[pl.ds(ebase + off, QCH)], src_v)
        pltpu.sync_copy(dst_hbm.at[pl.ds(ebase + off, QCH)], dst_v)
        for b in range(DEFER):
            _fire_g(b, b)
        for j in range(QCH):
            b = j % NBUF
            _wait_g(j, b)
            _fire_s(j, b)
            jn = j + DEFER
            if jn < QCH:
                if jn >= NBUF:
                    _wait_s(jn % NBUF)
                _fire_g(jn, jn % NBUF)
        for j in range(QCH - NBUF, QCH):
            _wait_s(j % NBUF)

    for q in range(4):
        _quarter(q * QCH)

    plsc.subcore_barrier()
    pltpu.sync_copy(acc.at[pl.ds(base, RPT)], o_hbm.at[w])


_sc_prop = pl.kernel(
    _prop_body,
    out_type=jax.ShapeDtypeStruct((NW, RPT, D), jnp.float32),
    mesh=plsc.VectorSubcoreMesh(core_axis_name="c", subcore_axis_name="s"),
    scratch_types=[
        pltpu.VMEM((QCH, CHUNK), jnp.int32),
        pltpu.VMEM((QCH, CHUNK), jnp.int32),
        pltpu.VMEM((CHUNK, D), jnp.float32),
        pltpu.VMEM((CHUNK, D), jnp.float32),
        pltpu.VMEM((CHUNK, D), jnp.float32),
        pltpu.VMEM_SHARED((ACC_ROWS, D), jnp.float32),
        pltpu.SemaphoreType.DMA,
        pltpu.SemaphoreType.DMA,
    ],
)


def _rspec(ncols=D):
    return pl.BlockSpec((ROW_BLK, ncols), lambda i: (i, 0))


def _dis_body(deg_ref, dis_ref, dis2_ref, dinv_ref):
    d = deg_ref[...] + 1.0
    r = lax.rsqrt(d)
    dis_ref[...] = r
    dis2_ref[...] = 1.0 / d
    dinv_ref[...] = jnp.sqrt(d)


def _dis_call(deg):
    return pl.pallas_call(
        _dis_body,
        grid=(N // ROW_BLK,),
        in_specs=[_rspec(1)],
        out_specs=[_rspec(1)] * 3,
        out_shape=[jax.ShapeDtypeStruct((N, 1), jnp.float32)] * 3,
    )(deg)


def _mm1_body(x_ref, w_ref, dis_ref, o_ref):
    t = jnp.dot(x_ref[...], w_ref[...], preferred_element_type=jnp.float32)
    o_ref[...] = t * dis_ref[...]


def _mm1_call(x, W1, dis):
    return pl.pallas_call(
        _mm1_body,
        grid=(N // ROW_BLK,),
        in_specs=[_rspec(), pl.BlockSpec((D, D), lambda i: (0, 0)), _rspec(1)],
        out_specs=_rspec(),
        out_shape=jax.ShapeDtypeStruct((N, D), jnp.float32),
    )(x, W1, dis)


def _first_body(a0_ref, a1_ref, t_ref, dis_ref, b1_ref, v_ref, w_ref):
    dis = dis_ref[...]
    h = jnp.maximum(dis * (a0_ref[...] + a1_ref[...] + t_ref[...])
                    + b1_ref[...], 0.0)
    v = dis * h
    v_ref[...] = v
    w_ref[...] = ALPHA * v


def _first_call(a0, a1, t, dis, b1):
    return pl.pallas_call(
        _first_body,
        grid=(N // ROW_BLK,),
        in_specs=[_rspec(), _rspec(), _rspec(), _rspec(1),
                  pl.BlockSpec((1, D), lambda i: (0, 0))],
        out_specs=[_rspec()] * 2,
        out_shape=[jax.ShapeDtypeStruct((N, D), jnp.float32)] * 2,
    )(a0, a1, t, dis, b1)


def _mix_body(a0_ref, a1_ref, v_ref, w_ref, dis2_ref, o_ref):
    f = (1.0 - ALPHA) * dis2_ref[...]
    o_ref[...] = f * (a0_ref[...] + a1_ref[...] + v_ref[...]) + w_ref[...]


def _mix_call(a0, a1, v, w, dis2):
    return pl.pallas_call(
        _mix_body,
        grid=(N // ROW_BLK,),
        in_specs=[_rspec(), _rspec(), _rspec(), _rspec(), _rspec(1)],
        out_specs=_rspec(),
        out_shape=jax.ShapeDtypeStruct((N, D), jnp.float32),
    )(a0, a1, v, w, dis2)


def _mm2_body(v_ref, w_ref, dinv_ref, dis_ref, o_ref):
    h = dinv_ref[...] * v_ref[...]
    g = jnp.dot(h, w_ref[...], preferred_element_type=jnp.float32)
    o_ref[...] = dis_ref[...] * g


def _mm2_call(v, W2, dinv, dis):
    return pl.pallas_call(
        _mm2_body,
        grid=(N // ROW_BLK,),
        in_specs=[_rspec(), pl.BlockSpec((D, D), lambda i: (0, 0)),
                  _rspec(1), _rspec(1)],
        out_specs=_rspec(),
        out_shape=jax.ShapeDtypeStruct((N, D), jnp.float32),
    )(v, W2, dinv, dis)


def _out_body(a0_ref, a1_ref, g_ref, dis_ref, b2_ref, o_ref):
    o = dis_ref[...] * (a0_ref[...] + a1_ref[...] + g_ref[...])
    o_ref[...] = o + b2_ref[...]


def _out_call(a0, a1, g, dis, b2):
    return pl.pallas_call(
        _out_body,
        grid=(N // ROW_BLK,),
        in_specs=[_rspec(), _rspec(), _rspec(), _rspec(1),
                  pl.BlockSpec((1, D), lambda i: (0, 0))],
        out_specs=_rspec(),
        out_shape=jax.ShapeDtypeStruct((N, D), jnp.float32),
    )(a0, a1, g, dis, b2)


def _halves(o):
    a0 = o[:NS].reshape(ACC_ROWS, D)[:N]
    a1 = o[NS:].reshape(ACC_ROWS, D)[:N]
    return a0, a1


def kernel(x, edge_index, W1, b1, W2, b2):
    pad = EPT * NW - E
    src = jnp.concatenate([edge_index[0], jnp.zeros((pad,), jnp.int32)])
    dst = jnp.concatenate([edge_index[1], jnp.full((pad,), TRASH, jnp.int32)])
    src_g = src.reshape(NW * NCHUNK, CHUNK)
    dst_g = dst.reshape(NW * NCHUNK, CHUNK)
    b1r = b1.reshape(1, D)
    b2r = b2.reshape(1, D)

    ones = jnp.ones((N, D), jnp.float32)
    d0, d1 = _halves(_sc_prop(src_g, dst_g, ones))
    deg = d0[:, 0:1] + d1[:, 0:1]
    dis, dis2, dinv = _dis_call(deg)

    t = _mm1_call(x, W1, dis)
    a0, a1 = _halves(_sc_prop(src_g, dst_g, t))
    v, w = _first_call(a0, a1, t, dis, b1r)

    for _ in range(K_ITERS):
        a0, a1 = _halves(_sc_prop(src_g, dst_g, v))
        v = _mix_call(a0, a1, v, w, dis2)

    g = _mm2_call(v, W2, dinv, dis)
    a0, a1 = _halves(_sc_prop(src_g, dst_g, g))
    return _out_call(a0, a1, g, dis, b2r)


# speedup vs baseline: 1.0829x; 1.0088x over previous
import jax
import jax.numpy as jnp
from jax import lax
from jax.experimental import pallas as pl
from jax.experimental.pallas import tpu as pltpu
from jax.experimental.pallas import tpu_sc as plsc

N = 10000
E = 320000
D = 128
K_ITERS = 10
ALPHA = 0.1

NC = 2
NS = 16
NW = NC * NS
CHUNK = 80
NCHUNK = 128
EPT = NCHUNK * CHUNK
RPT = 626
ACC_ROWS = NS * RPT
TRASH = N

ROW_BLK = 1000


NBUF = 3
DEFER = 2
HLF = NCHUNK // 2


def _zero_acc(stage0, acc, base):
    zf = jnp.zeros((16,), jnp.float32)

    def _zrow(i, carry):
        for jj in range(D // 16):
            stage0[i, pl.ds(jj * 16, 16)] = zf
        return carry

    lax.fori_loop(0, CHUNK, _zrow, 0)
    for k in range((RPT + CHUNK - 1) // CHUNK):
        nr = min(CHUNK, RPT - k * CHUNK)
        pltpu.sync_copy(stage0.at[pl.ds(0, nr)],
                        acc.at[pl.ds(base + k * CHUNK, nr)])


def _prop_body(src_hbm, dst_hbm, v_hbm, o_hbm,
               src_v, dst_v, stage0, stage1, stage2, acc, gsem, ssem):
    stage = (stage0, stage1, stage2)
    c = lax.axis_index("c")
    s = lax.axis_index("s")
    w = c * NS + s
    base = s * RPT

    _zero_acc(stage0, acc, base)
    plsc.subcore_barrier()

    def _fire_g(j, b):
        pltpu.async_copy(v_hbm.at[src_v.at[pl.ds(j * CHUNK, CHUNK)]],
                         stage[b], gsem)

    def _wait_g(j, b):
        pltpu.make_async_copy(v_hbm.at[src_v.at[pl.ds(j * CHUNK, CHUNK)]],
                              stage[b], gsem).wait()

    def _fire_s(j, b):
        pltpu.async_copy(stage[b], acc.at[dst_v.at[j]], ssem, add=True)

    def _wait_s(b):
        pltpu.make_async_copy(stage[b], acc.at[dst_v.at[0]], ssem).wait()

    def _half(off):
        pltpu.sync_copy(src_hbm.at[pl.ds(w * EPT + off * CHUNK, HLF * CHUNK)],
                        src_v)
        pltpu.sync_copy(dst_hbm.at[pl.ds(w * NCHUNK + off, HLF)], dst_v)
        for b in range(DEFER):
            _fire_g(b, b)
        for j in range(HLF):
            b = j % NBUF
            _wait_g(j, b)
            _fire_s(j, b)
            jn = j + DEFER
            if jn < HLF:
                if jn >= NBUF:
                    _wait_s(jn % NBUF)
                _fire_g(jn, jn % NBUF)
        for j in range(HLF - NBUF, HLF):
            _wait_s(j % NBUF)

    _half(0)
    _half(HLF)

    plsc.subcore_barrier()
    pltpu.sync_copy(acc.at[pl.ds(base, RPT)], o_hbm.at[w])


_sc_prop = pl.kernel(
    _prop_body,
    out_type=jax.ShapeDtypeStruct((NW, RPT, D), jnp.float32),
    mesh=plsc.VectorSubcoreMesh(core_axis_name="c", subcore_axis_name="s"),
    scratch_types=[
        pltpu.VMEM((HLF * CHUNK,), jnp.int32),
        pltpu.VMEM((HLF, CHUNK), jnp.int32),
        pltpu.VMEM((CHUNK, D), jnp.float32),
        pltpu.VMEM((CHUNK, D), jnp.float32),
        pltpu.VMEM((CHUNK, D), jnp.float32),
        pltpu.VMEM_SHARED((ACC_ROWS, D), jnp.float32),
        pltpu.SemaphoreType.DMA,
        pltpu.SemaphoreType.DMA,
    ],
)


def _deg_body(dst_hbm, o_hbm, dst_v, ones_v, zero_v, acc, ssem):
    c = lax.axis_index("c")
    s = lax.axis_index("s")
    w = c * NS + s
    base = s * RPT

    one = jnp.ones((16,), jnp.float32)

    def _orow(i, carry):
        for jj in range(D // 16):
            ones_v[i, pl.ds(jj * 16, 16)] = one
        return carry

    lax.fori_loop(0, CHUNK, _orow, 0)
    _zero_acc(zero_v, acc, base)
    plsc.subcore_barrier()

    pltpu.sync_copy(dst_hbm.at[pl.ds(w * NCHUNK, HLF)], dst_v)

    def _fire(j):
        pltpu.async_copy(ones_v, acc.at[dst_v.at[j]], ssem, add=True)

    def _wait():
        pltpu.make_async_copy(ones_v, acc.at[dst_v.at[0]], ssem).wait()

    def _deg_half(off):
        if off:
            pltpu.sync_copy(dst_hbm.at[pl.ds(w * NCHUNK + off, HLF)], dst_v)
        for j in range(HLF):
            _fire(j)
            if j >= NBUF:
                _wait()
        for _ in range(NBUF):
            _wait()

    _deg_half(0)
    _deg_half(HLF)

    plsc.subcore_barrier()
    pltpu.sync_copy(acc.at[pl.ds(base, RPT)], o_hbm.at[w])


_sc_deg = pl.kernel(
    _deg_body,
    out_type=jax.ShapeDtypeStruct((NW, RPT, D), jnp.float32),
    mesh=plsc.VectorSubcoreMesh(core_axis_name="c", subcore_axis_name="s"),
    scratch_types=[
        pltpu.VMEM((HLF, CHUNK), jnp.int32),
        pltpu.VMEM((CHUNK, D), jnp.float32),
        pltpu.VMEM((CHUNK, D), jnp.float32),
        pltpu.VMEM_SHARED((ACC_ROWS, D), jnp.float32),
        pltpu.SemaphoreType.DMA,
    ],
)


def _rspec(ncols=D):
    return pl.BlockSpec((ROW_BLK, ncols), lambda i: (i, 0))


def _dis_body(deg_ref, dis_ref, dis2_ref, dinv_ref):
    d = deg_ref[...] + 1.0
    r = lax.rsqrt(d)
    dis_ref[...] = r
    dis2_ref[...] = 1.0 / d
    dinv_ref[...] = jnp.sqrt(d)


def _dis_call(deg):
    return pl.pallas_call(
        _dis_body,
        grid=(N // ROW_BLK,),
        in_specs=[_rspec(1)],
        out_specs=[_rspec(1)] * 3,
        out_shape=[jax.ShapeDtypeStruct((N, 1), jnp.float32)] * 3,
    )(deg)


def _mm1_body(x_ref, w_ref, dis_ref, o_ref):
    t = jnp.dot(x_ref[...], w_ref[...], preferred_element_type=jnp.float32)
    o_ref[...] = t * dis_ref[...]


def _mm1_call(x, W1, dis):
    return pl.pallas_call(
        _mm1_body,
        grid=(N // ROW_BLK,),
        in_specs=[_rspec(), pl.BlockSpec((D, D), lambda i: (0, 0)), _rspec(1)],
        out_specs=_rspec(),
        out_shape=jax.ShapeDtypeStruct((N, D), jnp.float32),
    )(x, W1, dis)


def _first_body(a0_ref, a1_ref, t_ref, dis_ref, b1_ref, v_ref, w_ref):
    dis = dis_ref[...]
    h = jnp.maximum(dis * (a0_ref[...] + a1_ref[...] + t_ref[...])
                    + b1_ref[...], 0.0)
    v = dis * h
    v_ref[...] = v
    w_ref[...] = ALPHA * v


def _first_call(a0, a1, t, dis, b1):
    return pl.pallas_call(
        _first_body,
        grid=(N // ROW_BLK,),
        in_specs=[_rspec(), _rspec(), _rspec(), _rspec(1),
                  pl.BlockSpec((1, D), lambda i: (0, 0))],
        out_specs=[_rspec()] * 2,
        out_shape=[jax.ShapeDtypeStruct((N, D), jnp.float32)] * 2,
    )(a0, a1, t, dis, b1)


def _mix_body(a0_ref, a1_ref, v_ref, w_ref, dis2_ref, o_ref):
    f = (1.0 - ALPHA) * dis2_ref[...]
    o_ref[...] = f * (a0_ref[...] + a1_ref[...] + v_ref[...]) + w_ref[...]


def _mix_call(a0, a1, v, w, dis2):
    return pl.pallas_call(
        _mix_body,
        grid=(N // ROW_BLK,),
        in_specs=[_rspec(), _rspec(), _rspec(), _rspec(), _rspec(1)],
        out_specs=_rspec(),
        out_shape=jax.ShapeDtypeStruct((N, D), jnp.float32),
    )(a0, a1, v, w, dis2)


def _mm2_body(v_ref, w_ref, dinv_ref, dis_ref, o_ref):
    h = dinv_ref[...] * v_ref[...]
    g = jnp.dot(h, w_ref[...], preferred_element_type=jnp.float32)
    o_ref[...] = dis_ref[...] * g


def _mm2_call(v, W2, dinv, dis):
    return pl.pallas_call(
        _mm2_body,
        grid=(N // ROW_BLK,),
        in_specs=[_rspec(), pl.BlockSpec((D, D), lambda i: (0, 0)),
                  _rspec(1), _rspec(1)],
        out_specs=_rspec(),
        out_shape=jax.ShapeDtypeStruct((N, D), jnp.float32),
    )(v, W2, dinv, dis)


def _out_body(a0_ref, a1_ref, g_ref, dis_ref, b2_ref, o_ref):
    o = dis_ref[...] * (a0_ref[...] + a1_ref[...] + g_ref[...])
    o_ref[...] = o + b2_ref[...]


def _out_call(a0, a1, g, dis, b2):
    return pl.pallas_call(
        _out_body,
        grid=(N // ROW_BLK,),
        in_specs=[_rspec(), _rspec(), _rspec(), _rspec(1),
                  pl.BlockSpec((1, D), lambda i: (0, 0))],
        out_specs=_rspec(),
        out_shape=jax.ShapeDtypeStruct((N, D), jnp.float32),
    )(a0, a1, g, dis, b2)


def _halves(o):
    a0 = o[:NS].reshape(ACC_ROWS, D)[:N]
    a1 = o[NS:].reshape(ACC_ROWS, D)[:N]
    return a0, a1


def kernel(x, edge_index, W1, b1, W2, b2):
    pad = EPT * NW - E
    src = jnp.concatenate([edge_index[0], jnp.zeros((pad,), jnp.int32)])
    dst = jnp.concatenate([edge_index[1], jnp.full((pad,), TRASH, jnp.int32)])
    src_g = src
    dst_g = dst.reshape(NW * NCHUNK, CHUNK)
    b1r = b1.reshape(1, D)
    b2r = b2.reshape(1, D)

    d0, d1 = _halves(_sc_deg(dst_g))
    deg = d0[:, 0:1] + d1[:, 0:1]
    dis, dis2, dinv = _dis_call(deg)

    t = _mm1_call(x, W1, dis)
    a0, a1 = _halves(_sc_prop(src_g, dst_g, t))
    v, w = _first_call(a0, a1, t, dis, b1r)

    for _ in range(K_ITERS):
        a0, a1 = _halves(_sc_prop(src_g, dst_g, v))
        v = _mix_call(a0, a1, v, w, dis2)

    g = _mm2_call(v, W2, dinv, dis)
    a0, a1 = _halves(_sc_prop(src_g, dst_g, g))
    return _out_call(a0, a1, g, dis, b2r)


# speedup vs baseline: 1.0831x; 1.0002x over previous
import jax
import jax.numpy as jnp
from jax import lax
from jax.experimental import pallas as pl
from jax.experimental.pallas import tpu as pltpu
from jax.experimental.pallas import tpu_sc as plsc

N = 10000
E = 320000
D = 128
K_ITERS = 10
ALPHA = 0.1

NC = 2
NS = 16
NW = NC * NS
CHUNK = 80
NCHUNK = 128
EPT = NCHUNK * CHUNK
RPT = 626
ACC_ROWS = NS * RPT
TRASH = N

ROW_BLK = 1000


NBUF = 3
DEFER = 2
HLF = NCHUNK // 2


def _zero_acc(stage0, acc, base):
    zf = jnp.zeros((16,), jnp.float32)

    def _zrow(i, carry):
        for jj in range(D // 16):
            stage0[i, pl.ds(jj * 16, 16)] = zf
        return carry

    lax.fori_loop(0, CHUNK, _zrow, 0)
    for k in range((RPT + CHUNK - 1) // CHUNK):
        nr = min(CHUNK, RPT - k * CHUNK)
        pltpu.sync_copy(stage0.at[pl.ds(0, nr)],
                        acc.at[pl.ds(base + k * CHUNK, nr)])


def _prop_body(src_hbm, dst_hbm, v_hbm, o_hbm,
               src_v, dst_v, stage0, stage1, stage2, acc, gsem, ssem):
    stage = (stage0, stage1, stage2)
    c = lax.axis_index("c")
    s = lax.axis_index("s")
    w = c * NS + s
    base = s * RPT

    _zero_acc(stage0, acc, base)
    plsc.subcore_barrier()

    def _fire_g(j, b):
        pltpu.async_copy(v_hbm.at[src_v.at[pl.ds(j * CHUNK, CHUNK)]],
                         stage[b], gsem)

    def _wait_g(j, b):
        pltpu.make_async_copy(v_hbm.at[src_v.at[pl.ds(j * CHUNK, CHUNK)]],
                              stage[b], gsem).wait()

    def _fire_s(j, b):
        pltpu.async_copy(stage[b], acc.at[dst_v.at[j]], ssem, add=True)

    def _wait_s(b):
        pltpu.make_async_copy(stage[b], acc.at[dst_v.at[0]], ssem).wait()

    def _half(off):
        pltpu.sync_copy(src_hbm.at[pl.ds(w * EPT + off * CHUNK, HLF * CHUNK)],
                        src_v)
        pltpu.sync_copy(dst_hbm.at[pl.ds(w * NCHUNK + off, HLF)], dst_v)
        for b in range(DEFER):
            _fire_g(b, b)
        for j in range(HLF):
            b = j % NBUF
            _wait_g(j, b)
            _fire_s(j, b)
            jn = j + DEFER
            if jn < HLF:
                if jn >= NBUF:
                    _wait_s(jn % NBUF)
                _fire_g(jn, jn % NBUF)
        for j in range(HLF - NBUF, HLF):
            _wait_s(j % NBUF)

    _half(0)
    _half(HLF)

    plsc.subcore_barrier()
    pltpu.sync_copy(acc.at[pl.ds(base, RPT)], o_hbm.at[w])


_sc_prop = pl.kernel(
    _prop_body,
    out_type=jax.ShapeDtypeStruct((NW, RPT, D), jnp.float32),
    mesh=plsc.VectorSubcoreMesh(core_axis_name="c", subcore_axis_name="s"),
    scratch_types=[
        pltpu.VMEM((HLF * CHUNK,), jnp.int32),
        pltpu.VMEM((HLF, CHUNK), jnp.int32),
        pltpu.VMEM((CHUNK, D), jnp.float32),
        pltpu.VMEM((CHUNK, D), jnp.float32),
        pltpu.VMEM((CHUNK, D), jnp.float32),
        pltpu.VMEM_SHARED((ACC_ROWS, D), jnp.float32),
        pltpu.SemaphoreType.DMA,
        pltpu.SemaphoreType.DMA,
    ],
)


def _deg_body(dst_hbm, o_hbm, dst_v, ones_v, zero_v, acc, ssem):
    c = lax.axis_index("c")
    s = lax.axis_index("s")
    w = c * NS + s
    base = s * RPT

    one = jnp.ones((16,), jnp.float32)

    def _orow(i, carry):
        for jj in range(D // 16):
            ones_v[i, pl.ds(jj * 16, 16)] = one
        return carry

    lax.fori_loop(0, CHUNK, _orow, 0)
    _zero_acc(zero_v, acc, base)
    plsc.subcore_barrier()

    pltpu.sync_copy(dst_hbm.at[pl.ds(w * NCHUNK, HLF)], dst_v)

    def _fire(j):
        pltpu.async_copy(ones_v, acc.at[dst_v.at[j]], ssem, add=True)

    def _wait():
        pltpu.make_async_copy(ones_v, acc.at[dst_v.at[0]], ssem).wait()

    DEG_LAG = 8

    def _deg_half(off):
        if off:
            pltpu.sync_copy(dst_hbm.at[pl.ds(w * NCHUNK + off, HLF)], dst_v)
        for j in range(HLF):
            _fire(j)
            if j >= DEG_LAG:
                _wait()
        for _ in range(DEG_LAG):
            _wait()

    _deg_half(0)
    _deg_half(HLF)

    plsc.subcore_barrier()
    pltpu.sync_copy(acc.at[pl.ds(base, RPT)], o_hbm.at[w])


_sc_deg = pl.kernel(
    _deg_body,
    out_type=jax.ShapeDtypeStruct((NW, RPT, D), jnp.float32),
    mesh=plsc.VectorSubcoreMesh(core_axis_name="c", subcore_axis_name="s"),
    scratch_types=[
        pltpu.VMEM((HLF, CHUNK), jnp.int32),
        pltpu.VMEM((CHUNK, D), jnp.float32),
        pltpu.VMEM((CHUNK, D), jnp.float32),
        pltpu.VMEM_SHARED((ACC_ROWS, D), jnp.float32),
        pltpu.SemaphoreType.DMA,
    ],
)


def _rspec(ncols=D):
    return pl.BlockSpec((ROW_BLK, ncols), lambda i: (i, 0))


def _dis_body(deg_ref, dis_ref, dis2_ref, dinv_ref):
    d = deg_ref[...] + 1.0
    r = lax.rsqrt(d)
    dis_ref[...] = r
    dis2_ref[...] = 1.0 / d
    dinv_ref[...] = jnp.sqrt(d)


def _dis_call(deg):
    return pl.pallas_call(
        _dis_body,
        grid=(N // ROW_BLK,),
        in_specs=[_rspec(1)],
        out_specs=[_rspec(1)] * 3,
        out_shape=[jax.ShapeDtypeStruct((N, 1), jnp.float32)] * 3,
    )(deg)


def _mm1_body(x_ref, w_ref, dis_ref, o_ref):
    t = jnp.dot(x_ref[...], w_ref[...], preferred_element_type=jnp.float32)
    o_ref[...] = t * dis_ref[...]


def _mm1_call(x, W1, dis):
    return pl.pallas_call(
        _mm1_body,
        grid=(N // ROW_BLK,),
        in_specs=[_rspec(), pl.BlockSpec((D, D), lambda i: (0, 0)), _rspec(1)],
        out_specs=_rspec(),
        out_shape=jax.ShapeDtypeStruct((N, D), jnp.float32),
    )(x, W1, dis)


def _first_body(a0_ref, a1_ref, t_ref, dis_ref, b1_ref, v_ref, w_ref):
    dis = dis_ref[...]
    h = jnp.maximum(dis * (a0_ref[...] + a1_ref[...] + t_ref[...])
                    + b1_ref[...], 0.0)
    v = dis * h
    v_ref[...] = v
    w_ref[...] = ALPHA * v


def _first_call(a0, a1, t, dis, b1):
    return pl.pallas_call(
        _first_body,
        grid=(N // ROW_BLK,),
        in_specs=[_rspec(), _rspec(), _rspec(), _rspec(1),
                  pl.BlockSpec((1, D), lambda i: (0, 0))],
        out_specs=[_rspec()] * 2,
        out_shape=[jax.ShapeDtypeStruct((N, D), jnp.float32)] * 2,
    )(a0, a1, t, dis, b1)


def _mix_body(a0_ref, a1_ref, v_ref, w_ref, dis2_ref, o_ref):
    f = (1.0 - ALPHA) * dis2_ref[...]
    o_ref[...] = f * (a0_ref[...] + a1_ref[...] + v_ref[...]) + w_ref[...]


def _mix_call(a0, a1, v, w, dis2):
    return pl.pallas_call(
        _mix_body,
        grid=(N // ROW_BLK,),
        in_specs=[_rspec(), _rspec(), _rspec(), _rspec(), _rspec(1)],
        out_specs=_rspec(),
        out_shape=jax.ShapeDtypeStruct((N, D), jnp.float32),
    )(a0, a1, v, w, dis2)


def _mm2_body(v_ref, w_ref, dinv_ref, dis_ref, o_ref):
    h = dinv_ref[...] * v_ref[...]
    g = jnp.dot(h, w_ref[...], preferred_element_type=jnp.float32)
    o_ref[...] = dis_ref[...] * g


def _mm2_call(v, W2, dinv, dis):
    return pl.pallas_call(
        _mm2_body,
        grid=(N // ROW_BLK,),
        in_specs=[_rspec(), pl.BlockSpec((D, D), lambda i: (0, 0)),
                  _rspec(1), _rspec(1)],
        out_specs=_rspec(),
        out_shape=jax.ShapeDtypeStruct((N, D), jnp.float32),
    )(v, W2, dinv, dis)


def _out_body(a0_ref, a1_ref, g_ref, dis_ref, b2_ref, o_ref):
    o = dis_ref[...] * (a0_ref[...] + a1_ref[...] + g_ref[...])
    o_ref[...] = o + b2_ref[...]


def _out_call(a0, a1, g, dis, b2):
    return pl.pallas_call(
        _out_body,
        grid=(N // ROW_BLK,),
        in_specs=[_rspec(), _rspec(), _rspec(), _rspec(1),
                  pl.BlockSpec((1, D), lambda i: (0, 0))],
        out_specs=_rspec(),
        out_shape=jax.ShapeDtypeStruct((N, D), jnp.float32),
    )(a0, a1, g, dis, b2)


def _halves(o):
    a0 = o[:NS].reshape(ACC_ROWS, D)[:N]
    a1 = o[NS:].reshape(ACC_ROWS, D)[:N]
    return a0, a1


def kernel(x, edge_index, W1, b1, W2, b2):
    pad = EPT * NW - E
    src = jnp.concatenate([edge_index[0], jnp.zeros((pad,), jnp.int32)])
    dst = jnp.concatenate([edge_index[1], jnp.full((pad,), TRASH, jnp.int32)])
    src_g = src
    dst_g = dst.reshape(NW * NCHUNK, CHUNK)
    b1r = b1.reshape(1, D)
    b2r = b2.reshape(1, D)

    d0, d1 = _halves(_sc_deg(dst_g))
    deg = d0[:, 0:1] + d1[:, 0:1]
    dis, dis2, dinv = _dis_call(deg)

    t = _mm1_call(x, W1, dis)
    a0, a1 = _halves(_sc_prop(src_g, dst_g, t))
    v, w = _first_call(a0, a1, t, dis, b1r)

    for _ in range(K_ITERS):
        a0, a1 = _halves(_sc_prop(src_g, dst_g, v))
        v = _mix_call(a0, a1, v, w, dis2)

    g = _mm2_call(v, W2, dinv, dis)
    a0, a1 = _halves(_sc_prop(src_g, dst_g, g))
    return _out_call(a0, a1, g, dis, b2r)
